# baseline (speedup 1.0000x reference)
import jax
import jax.numpy as jnp
from jax.experimental import pallas as pl


def _pos_kernel(row_ref, col_ref, out_ref):
    d = row_ref.shape[1]
    h = row_ref.shape[0]
    w = col_ref.shape[0]
    col_t = col_ref[...].T
    row_t = row_ref[...].T
    x = jnp.broadcast_to(col_t[:, None, :], (d, h, w)).reshape(d, h * w)
    y = jnp.broadcast_to(row_t[:, :, None], (d, h, w)).reshape(d, h * w)
    out_ref[0, :d, :] = x
    out_ref[0, d:, :] = y


def kernel(pixel_values, row_embedding, column_embedding):
    b = pixel_values.shape[0]
    h, w = pixel_values.shape[-2], pixel_values.shape[-1]
    d = row_embedding.shape[-1]
    row = row_embedding[:h]
    col = column_embedding[:w]
    out = pl.pallas_call(
        _pos_kernel,
        grid=(b,),
        in_specs=[
            pl.BlockSpec((h, d), lambda i: (0, 0)),
            pl.BlockSpec((w, d), lambda i: (0, 0)),
        ],
        out_specs=pl.BlockSpec((1, 2 * d, h * w), lambda i: (i, 0, 0)),
        out_shape=jax.ShapeDtypeStruct((b, 2 * d, h * w), jnp.float32),
    )(row, col)
    return out.reshape(b, 2 * d, h, w)


# speedup vs baseline: 1.5436x; 1.5436x over previous
import jax
import jax.numpy as jnp
from jax.experimental import pallas as pl
from jax.experimental.pallas import tpu as pltpu


def _pos_kernel(row_ref, col_ref, out_ref, pat_ref, sems):
    h, d = row_ref.shape
    w = col_ref.shape[0]
    hw = h * w
    b = out_ref.shape[0]
    lane = jax.lax.broadcasted_iota(jnp.int32, (w, hw), 1)
    sub_w = jax.lax.broadcasted_iota(jnp.int32, (w, hw), 0)
    sx = jnp.where(lane % w == sub_w, 1.0, 0.0).astype(jnp.float32)
    lane_h = jax.lax.broadcasted_iota(jnp.int32, (h, hw), 1)
    sub_h = jax.lax.broadcasted_iota(jnp.int32, (h, hw), 0)
    sy = jnp.where(lane_h // w == sub_h, 1.0, 0.0).astype(jnp.float32)
    dn = (((0,), (0,)), ((), ()))
    pat_ref[:d, :] = jax.lax.dot_general(
        col_ref[...], sx, dn, preferred_element_type=jnp.float32)
    pat_ref[d:, :] = jax.lax.dot_general(
        row_ref[...], sy, dn, preferred_element_type=jnp.float32)
    copies = [
        pltpu.make_async_copy(pat_ref, out_ref.at[i], sems.at[i])
        for i in range(b)
    ]
    for c in copies:
        c.start()
    for c in copies:
        c.wait()


def kernel(pixel_values, row_embedding, column_embedding):
    b = pixel_values.shape[0]
    h, w = pixel_values.shape[-2], pixel_values.shape[-1]
    d = row_embedding.shape[-1]
    row = row_embedding[:h]
    col = column_embedding[:w]
    out = pl.pallas_call(
        _pos_kernel,
        in_specs=[
            pl.BlockSpec((h, d), lambda: (0, 0)),
            pl.BlockSpec((w, d), lambda: (0, 0)),
        ],
        out_specs=pl.BlockSpec(memory_space=pl.ANY),
        out_shape=jax.ShapeDtypeStruct((b, 2 * d, h * w), jnp.float32),
        scratch_shapes=[
            pltpu.VMEM((2 * d, h * w), jnp.float32),
            pltpu.SemaphoreType.DMA((b,)),
        ],
    )(row, col)
    return out.reshape(b, 2 * d, h, w)
